# baseline (speedup 1.0000x reference)
import functools

import jax
import jax.numpy as jnp
from jax import lax
from jax.experimental import pallas as pl
from jax.experimental.pallas import tpu as pltpu
from jax.experimental.pallas import tpu_sc as plsc

_B = 1024
_C = 100000
_W = 2048
_NBLK = (_C + _W - 1) // _W

_NC = 2
_NS = 16
_NW = _NC * _NS
_BPW = _B // _NW
_L = 16


def _sc_gather_body(costh_hbm, label_hbm, g_hbm, lab_v, idx_v, g_v, sem):
    wid = lax.axis_index("s") * _NC + lax.axis_index("c")
    base = wid * _BPW
    pltpu.sync_copy(label_hbm.at[pl.ds(base, _BPW)], lab_v)
    for j in range(_BPW // _L):
        lab16 = lab_v[pl.ds(j * _L, _L)]
        rows = base + j * _L + lax.broadcasted_iota(jnp.int32, (_L,), 0)
        idx_v[pl.ds(j * _L, _L)] = rows * _C + lab16
    pltpu.async_copy(costh_hbm.at[idx_v], g_v, sem).wait()
    pltpu.sync_copy(g_v, g_hbm.at[pl.ds(base, _BPW)])


_sc_gather = functools.partial(
    pl.kernel,
    out_type=jax.ShapeDtypeStruct((_B,), jnp.float32),
    mesh=plsc.VectorSubcoreMesh(core_axis_name="c", subcore_axis_name="s"),
    scratch_types=[
        pltpu.VMEM((_BPW,), jnp.int32),
        pltpu.VMEM((_BPW,), jnp.int32),
        pltpu.VMEM((_BPW,), jnp.float32),
        pltpu.SemaphoreType.DMA,
    ],
)(_sc_gather_body)


def _loss_kernel(costh_ref, g_ref, out_ref, se_acc):
    jb = pl.program_id(0)

    @pl.when(jb == 0)
    def _init():
        se_acc[...] = jnp.zeros_like(se_acc)

    c = costh_ref[...]

    @pl.when(jb < _NBLK - 1)
    def _main():
        se_acc[...] += jnp.sum(jnp.exp(8.5 * c), axis=1, keepdims=True)

    @pl.when(jb == _NBLK - 1)
    def _fin():
        cols = jb * _W + lax.broadcasted_iota(jnp.int32, (_B, _W), 1)
        e = jnp.where(cols < _C, jnp.exp(8.5 * c), 0.0)
        se = se_acc[...] + jnp.sum(e, axis=1, keepdims=True)
        a_g = 8.5 * g_ref[...]
        picked = a_g - 2.25
        se = se - jnp.exp(a_g) + jnp.exp(picked)
        loss_i = jnp.log(se) - picked
        out_ref[...] = jnp.mean(loss_i, keepdims=True)


def kernel(costh, label):
    g = _sc_gather(costh.reshape(_B * _C), label.astype(jnp.int32))
    out = pl.pallas_call(
        _loss_kernel,
        grid=(_NBLK,),
        in_specs=[
            pl.BlockSpec((_B, _W), lambda j: (0, j)),
            pl.BlockSpec((_B, 1), lambda j: (0, 0)),
        ],
        out_specs=pl.BlockSpec((1, 1), lambda j: (0, 0)),
        out_shape=jax.ShapeDtypeStruct((1, 1), jnp.float32),
        scratch_shapes=[pltpu.VMEM((_B, 1), jnp.float32)],
    )(costh, g.reshape(_B, 1))
    return out[0, 0]


# speedup vs baseline: 2.1667x; 2.1667x over previous
import jax
import jax.numpy as jnp
from jax import lax
from jax.experimental import pallas as pl
from jax.experimental.pallas import tpu as pltpu

_B = 1024
_C = 100000
_W = 2048
_NBLK = (_C + _W - 1) // _W


def _loss_kernel(costh_ref, label_ref, out_ref, se_acc, g_acc):
    jb = pl.program_id(0)

    @pl.when(jb == 0)
    def _init():
        se_acc[...] = jnp.zeros_like(se_acc)
        g_acc[...] = jnp.zeros_like(g_acc)

    c = costh_ref[...]
    lrel = label_ref[...] - jb * _W
    is_lab = lax.broadcasted_iota(jnp.int32, (_B, _W), 1) == lrel
    g_acc[...] += jnp.sum(jnp.where(is_lab, c, 0.0), axis=1, keepdims=True)

    @pl.when(jb < _NBLK - 1)
    def _main():
        se_acc[...] += jnp.sum(jnp.exp(8.5 * c), axis=1, keepdims=True)

    @pl.when(jb == _NBLK - 1)
    def _fin():
        e = jnp.where(
            lax.broadcasted_iota(jnp.int32, (_B, _W), 1) < (_C - jb * _W),
            jnp.exp(8.5 * c), 0.0)
        se = se_acc[...] + jnp.sum(e, axis=1, keepdims=True)
        a_g = 8.5 * g_acc[...]
        picked = a_g - 2.25
        se = se - jnp.exp(a_g) + jnp.exp(picked)
        loss_i = jnp.log(se) - picked
        out_ref[...] = jnp.mean(loss_i, keepdims=True)


def kernel(costh, label):
    label2d = label.astype(jnp.int32).reshape(_B, 1)
    out = pl.pallas_call(
        _loss_kernel,
        grid=(_NBLK,),
        in_specs=[
            pl.BlockSpec((_B, _W), lambda j: (0, j)),
            pl.BlockSpec((_B, 1), lambda j: (0, 0)),
        ],
        out_specs=pl.BlockSpec((1, 1), lambda j: (0, 0)),
        out_shape=jax.ShapeDtypeStruct((1, 1), jnp.float32),
        scratch_shapes=[
            pltpu.VMEM((_B, 1), jnp.float32),
            pltpu.VMEM((_B, 1), jnp.float32),
        ],
    )(costh, label2d)
    return out[0, 0]


# speedup vs baseline: 2.2197x; 1.0245x over previous
import jax
import jax.numpy as jnp
from jax import lax
from jax.experimental import pallas as pl
from jax.experimental.pallas import tpu as pltpu

_B = 1024
_C = 100000
_W = 4096
_NBLK = (_C + _W - 1) // _W


def _loss_kernel(costh_ref, label_ref, out_ref, se_acc, g_acc):
    jb = pl.program_id(0)

    @pl.when(jb == 0)
    def _init():
        se_acc[...] = jnp.zeros_like(se_acc)
        g_acc[...] = jnp.zeros_like(g_acc)

    c = costh_ref[...]
    lrel = label_ref[...] - jb * _W
    is_lab = lax.broadcasted_iota(jnp.int32, (_B, _W), 1) == lrel
    g_acc[...] += jnp.sum(jnp.where(is_lab, c, 0.0), axis=1, keepdims=True)

    @pl.when(jb < _NBLK - 1)
    def _main():
        se_acc[...] += jnp.sum(jnp.exp(8.5 * c), axis=1, keepdims=True)

    @pl.when(jb == _NBLK - 1)
    def _fin():
        e = jnp.where(
            lax.broadcasted_iota(jnp.int32, (_B, _W), 1) < (_C - jb * _W),
            jnp.exp(8.5 * c), 0.0)
        se = se_acc[...] + jnp.sum(e, axis=1, keepdims=True)
        a_g = 8.5 * g_acc[...]
        picked = a_g - 2.25
        se = se - jnp.exp(a_g) + jnp.exp(picked)
        loss_i = jnp.log(se) - picked
        out_ref[...] = jnp.mean(loss_i, keepdims=True)


def kernel(costh, label):
    label2d = label.astype(jnp.int32).reshape(_B, 1)
    out = pl.pallas_call(
        _loss_kernel,
        grid=(_NBLK,),
        in_specs=[
            pl.BlockSpec((_B, _W), lambda j: (0, j)),
            pl.BlockSpec((_B, 1), lambda j: (0, 0)),
        ],
        out_specs=pl.BlockSpec((1, 1), lambda j: (0, 0)),
        out_shape=jax.ShapeDtypeStruct((1, 1), jnp.float32),
        scratch_shapes=[
            pltpu.VMEM((_B, 1), jnp.float32),
            pltpu.VMEM((_B, 1), jnp.float32),
        ],
    )(costh, label2d)
    return out[0, 0]
